# baseline (speedup 1.0000x reference)
import functools

import jax
import jax.numpy as jnp
from jax import lax
from jax.experimental import pallas as pl
from jax.experimental.pallas import tpu as pltpu
from jax.experimental.pallas import tpu_sc as plsc

N = 10000
E = 320000
D = 128
H = 256
C = 47
CP = 48

NPAD = 10240
NTILE = 32
EPAD = 327680
EPT = EPAD // NTILE
BB = 128
KB = EPT // BB
RPS = NPAD // 16

_mesh = plsc.VectorSubcoreMesh(core_axis_name="c", subcore_axis_name="s")


def _zero_vmem_2d(zbuf, rows, cols):
    z16 = jnp.zeros((16,), jnp.float32)
    for r in range(rows):
        for c in range(cols // 16):
            zbuf[r, pl.ds(c * 16, 16)] = z16


def _unpack_batch(packed, j, sbuf, dbuf):
    mask = jnp.full((16,), 0x3FFF, jnp.int32)
    for k in range(BB // 16):
        v = packed[j, pl.ds(k * 16, 16)]
        sbuf[pl.ds(k * 16, 16)] = v & mask
        dbuf[pl.ds(k * 16, 16)] = lax.shift_right_logical(v, 14)


@functools.partial(
    pl.kernel,
    mesh=_mesh,
    out_type=(
        jax.ShapeDtypeStruct((NPAD,), jnp.float32),
        jax.ShapeDtypeStruct((NPAD,), jnp.float32),
    ),
    scratch_types=[
        pltpu.VMEM((KB, BB), jnp.int32),
        pltpu.VMEM((BB,), jnp.float32),
        pltpu.VMEM((RPS,), jnp.float32),
        pltpu.VMEM_SHARED((NPAD,), jnp.float32),
        pltpu.SemaphoreType.DMA,
    ],
)
def _deg_kernel(dst_hbm, out0, out1, didx, ones_v, zrow, acc, sem):
    cid = lax.axis_index("c")
    sid = lax.axis_index("s")
    wid = cid * 16 + sid
    for i in range(BB // 16):
        ones_v[pl.ds(i * 16, 16)] = jnp.ones((16,), jnp.float32)
    for i in range(RPS // 16):
        zrow[pl.ds(i * 16, 16)] = jnp.zeros((16,), jnp.float32)
    pltpu.sync_copy(dst_hbm.at[wid], didx)
    pltpu.sync_copy(zrow, acc.at[pl.ds(sid * RPS, RPS)])
    plsc.subcore_barrier()

    @pl.loop(0, KB)
    def _(j):
        pltpu.async_copy(ones_v, acc.at[didx.at[j]], sem, add=True)

    @pl.loop(0, KB)
    def _(j):
        pltpu.make_async_copy(ones_v, acc.at[didx.at[j]], sem).wait()

    plsc.subcore_barrier()

    @pl.when(cid == 0)
    def _():
        pltpu.sync_copy(acc.at[pl.ds(sid * RPS, RPS)],
                        out0.at[pl.ds(sid * RPS, RPS)])

    @pl.when(cid == 1)
    def _():
        pltpu.sync_copy(acc.at[pl.ds(sid * RPS, RPS)],
                        out1.at[pl.ds(sid * RPS, RPS)])


def _make_agg_kernel(width):
    @functools.partial(
        pl.kernel,
        mesh=_mesh,
        out_type=(
            jax.ShapeDtypeStruct((NPAD, width), jnp.float32),
            jax.ShapeDtypeStruct((NPAD, width), jnp.float32),
        ),
        scratch_types=[
            pltpu.VMEM((KB, BB), jnp.int32),
            [pltpu.VMEM((BB,), jnp.int32)] * 2,
            [pltpu.VMEM((BB,), jnp.int32)] * 2,
            [pltpu.VMEM((BB, width), jnp.float32)] * 2,
            pltpu.VMEM((16, width), jnp.float32),
            pltpu.VMEM_SHARED((NPAD, width), jnp.float32),
            [pltpu.SemaphoreType.DMA] * 2,
        ],
        compiler_params=pltpu.CompilerParams(use_tc_tiling_on_sc=False),
    )
    def agg(pidx_hbm, x_hbm, out0, out1, pidx, sbuf, dbuf, rows, zbuf,
            acc, gsem):
        cid = lax.axis_index("c")
        sid = lax.axis_index("s")
        wid = cid * 16 + sid
        _zero_vmem_2d(zbuf, 16, width)
        for t in range(RPS // 16):
            pltpu.sync_copy(zbuf, acc.at[pl.ds(sid * RPS + t * 16, 16)])
        pltpu.sync_copy(pidx_hbm.at[wid], pidx)
        plsc.subcore_barrier()

        def gath(j, b):
            _unpack_batch(pidx, j, sbuf[b], dbuf[b])
            pltpu.async_copy(x_hbm.at[sbuf[b]], rows[b], gsem[b])

        def scat(j, b):
            pltpu.make_async_copy(x_hbm.at[sbuf[b]], rows[b], gsem[b]).wait()
            pltpu.sync_copy(rows[b], acc.at[dbuf[b]], add=True)

        gath(0, 0)

        @pl.loop(0, KB // 2 - 1)
        def _(g):
            j0 = g * 2
            gath(j0 + 1, 1)
            scat(j0, 0)
            gath(j0 + 2, 0)
            scat(j0 + 1, 1)

        gath(KB - 1, 1)
        scat(KB - 2, 0)
        scat(KB - 1, 1)

        plsc.subcore_barrier()

        @pl.when(cid == 0)
        def _():
            pltpu.sync_copy(acc.at[pl.ds(sid * RPS, RPS)],
                            out0.at[pl.ds(sid * RPS, RPS)])

        @pl.when(cid == 1)
        def _():
            pltpu.sync_copy(acc.at[pl.ds(sid * RPS, RPS)],
                            out1.at[pl.ds(sid * RPS, RPS)])

    return agg


_agg_d = _make_agg_kernel(D)
_agg_c = _make_agg_kernel(CP)


_R = 512
_GRID = NPAD // _R


def _scale_in_body(x_ref, d0_ref, d1_ref, xt_ref, rdeg_ref):
    deg = jnp.maximum(d0_ref[...] + d1_ref[...], 1.0)
    rd = lax.rsqrt(deg)
    rdeg_ref[...] = rd
    xt_ref[...] = x_ref[...] * rd


def _scale_in(x_pad, deg0, deg1):
    return pl.pallas_call(
        _scale_in_body,
        grid=(_GRID,),
        in_specs=[
            pl.BlockSpec((_R, D), lambda i: (i, 0)),
            pl.BlockSpec((_R, 1), lambda i: (i, 0)),
            pl.BlockSpec((_R, 1), lambda i: (i, 0)),
        ],
        out_specs=[
            pl.BlockSpec((_R, D), lambda i: (i, 0)),
            pl.BlockSpec((_R, 1), lambda i: (i, 0)),
        ],
        out_shape=[
            jax.ShapeDtypeStruct((NPAD, D), jnp.float32),
            jax.ShapeDtypeStruct((NPAD, 1), jnp.float32),
        ],
    )(x_pad, deg0, deg1)


def _mid_body(a0_ref, a1_ref, rd_ref, w1_ref, b1_ref, w2_ref, yt_ref):
    rd = rd_ref[...]
    a = (a0_ref[...] + a1_ref[...]) * rd
    z = jnp.dot(a, w1_ref[...], preferred_element_type=jnp.float32)
    z = z + b1_ref[...]
    h = jnp.where(z > 0, z, jnp.exp(z) - 1.0)
    yt_ref[...] = jnp.dot(h * rd, w2_ref[...],
                          preferred_element_type=jnp.float32)


def _mid(a0, a1, rdeg, W1, b1, W2p):
    return pl.pallas_call(
        _mid_body,
        grid=(_GRID,),
        in_specs=[
            pl.BlockSpec((_R, D), lambda i: (i, 0)),
            pl.BlockSpec((_R, D), lambda i: (i, 0)),
            pl.BlockSpec((_R, 1), lambda i: (i, 0)),
            pl.BlockSpec((D, H), lambda i: (0, 0)),
            pl.BlockSpec((1, H), lambda i: (0, 0)),
            pl.BlockSpec((H, CP), lambda i: (0, 0)),
        ],
        out_specs=pl.BlockSpec((_R, CP), lambda i: (i, 0)),
        out_shape=jax.ShapeDtypeStruct((NPAD, CP), jnp.float32),
    )(a0, a1, rdeg, W1, b1, W2p)


def _scale_out_body(q0_ref, q1_ref, rd_ref, b2_ref, out_ref):
    out_ref[...] = (q0_ref[...] + q1_ref[...]) * rd_ref[...] + b2_ref[...]


def _scale_out(q0, q1, rdeg, b2p):
    return pl.pallas_call(
        _scale_out_body,
        grid=(_GRID,),
        in_specs=[
            pl.BlockSpec((_R, CP), lambda i: (i, 0)),
            pl.BlockSpec((_R, CP), lambda i: (i, 0)),
            pl.BlockSpec((_R, 1), lambda i: (i, 0)),
            pl.BlockSpec((1, CP), lambda i: (0, 0)),
        ],
        out_specs=pl.BlockSpec((_R, CP), lambda i: (i, 0)),
        out_shape=jax.ShapeDtypeStruct((NPAD, CP), jnp.float32),
    )(q0, q1, rdeg, b2p)


@jax.jit
def kernel(features, edge_index, W1, b1, W2, b2):
    epad = EPAD - E
    src_flat = jnp.pad(edge_index[0], (0, epad), constant_values=NPAD - 1)
    dst_flat = jnp.pad(edge_index[1], (0, epad), constant_values=NPAD - 1)
    packed = (src_flat | (dst_flat << 14)).reshape(NTILE, KB, BB)
    dst32 = dst_flat.reshape(NTILE, KB, BB)
    x_pad = jnp.pad(features, ((0, NPAD - N), (0, 0)))
    W2p = jnp.pad(W2, ((0, 0), (0, CP - C)))
    b1r = b1.reshape(1, H)
    b2p = jnp.pad(b2, (0, CP - C)).reshape(1, CP)

    deg0, deg1 = _deg_kernel(dst32)
    xt, rdeg = _scale_in(x_pad, deg0.reshape(NPAD, 1), deg1.reshape(NPAD, 1))
    a0, a1 = _agg_d(packed, xt)
    yt = _mid(a0, a1, rdeg, W1, b1r, W2p)
    q0, q1 = _agg_c(packed, yt)
    out = _scale_out(q0, q1, rdeg, b2p)
    return out[:N, :C]


# speedup vs baseline: 2.7368x; 2.7368x over previous
import functools

import jax
import jax.numpy as jnp
from jax import lax
from jax.experimental import pallas as pl
from jax.experimental.pallas import tpu as pltpu
from jax.experimental.pallas import tpu_sc as plsc

N = 10000
E = 320000
D = 128
H = 256
C = 47
CP = 48

NPAD = 10240
NTILE = 32
EPAD = 327680
EPT = EPAD // NTILE
BB = 128
KB = EPT // BB
RPS = NPAD // 16

_mesh = plsc.VectorSubcoreMesh(core_axis_name="c", subcore_axis_name="s")


def _zero_vmem_2d(zbuf, rows, cols):
    z16 = jnp.zeros((16,), jnp.float32)
    for r in range(rows):
        for c in range(cols // 16):
            zbuf[r, pl.ds(c * 16, 16)] = z16


def _unpack_batch(packed, j, sbuf, dbuf):
    mask = jnp.full((16,), 0x3FFF, jnp.int32)
    for k in range(BB // 16):
        v = packed[j, pl.ds(k * 16, 16)]
        sbuf[pl.ds(k * 16, 16)] = v & mask
        dbuf[pl.ds(k * 16, 16)] = lax.shift_right_logical(v, 14)


@functools.partial(
    pl.kernel,
    mesh=_mesh,
    out_type=(
        jax.ShapeDtypeStruct((NPAD,), jnp.float32),
        jax.ShapeDtypeStruct((NPAD,), jnp.float32),
    ),
    scratch_types=[
        pltpu.VMEM((KB, BB), jnp.int32),
        pltpu.VMEM((BB,), jnp.float32),
        pltpu.VMEM((RPS,), jnp.float32),
        pltpu.VMEM_SHARED((NPAD,), jnp.float32),
        pltpu.SemaphoreType.DMA,
    ],
)
def _deg_kernel(dst_hbm, out0, out1, didx, ones_v, zrow, acc, sem):
    cid = lax.axis_index("c")
    sid = lax.axis_index("s")
    wid = cid * 16 + sid
    for i in range(BB // 16):
        ones_v[pl.ds(i * 16, 16)] = jnp.ones((16,), jnp.float32)
    for i in range(RPS // 16):
        zrow[pl.ds(i * 16, 16)] = jnp.zeros((16,), jnp.float32)
    pltpu.sync_copy(dst_hbm.at[wid], didx)
    pltpu.sync_copy(zrow, acc.at[pl.ds(sid * RPS, RPS)])
    plsc.subcore_barrier()

    @pl.loop(0, KB)
    def _(j):
        pltpu.async_copy(ones_v, acc.at[didx.at[j]], sem, add=True)

    @pl.loop(0, KB)
    def _(j):
        pltpu.make_async_copy(ones_v, acc.at[didx.at[j]], sem).wait()

    plsc.subcore_barrier()

    @pl.when(cid == 0)
    def _():
        pltpu.sync_copy(acc.at[pl.ds(sid * RPS, RPS)],
                        out0.at[pl.ds(sid * RPS, RPS)])

    @pl.when(cid == 1)
    def _():
        pltpu.sync_copy(acc.at[pl.ds(sid * RPS, RPS)],
                        out1.at[pl.ds(sid * RPS, RPS)])


def _make_agg_kernel(width):
    @functools.partial(
        pl.kernel,
        mesh=_mesh,
        out_type=(
            jax.ShapeDtypeStruct((NPAD, width), jnp.float32),
            jax.ShapeDtypeStruct((NPAD, width), jnp.float32),
        ),
        scratch_types=[
            pltpu.VMEM((KB, BB), jnp.int32),
            [pltpu.VMEM((BB,), jnp.int32)] * 2,
            [pltpu.VMEM((BB,), jnp.int32)] * 2,
            [pltpu.VMEM((BB, width), jnp.float32)] * 2,
            pltpu.VMEM((16, width), jnp.float32),
            pltpu.VMEM_SHARED((NPAD, width), jnp.float32),
            [pltpu.SemaphoreType.DMA] * 2,
        ],
        compiler_params=pltpu.CompilerParams(use_tc_tiling_on_sc=False),
    )
    def agg(pidx_hbm, x_hbm, out0, out1, pidx, sbuf, dbuf, rows, zbuf,
            acc, gsem):
        cid = lax.axis_index("c")
        sid = lax.axis_index("s")
        wid = cid * 16 + sid
        _zero_vmem_2d(zbuf, 16, width)
        for t in range(RPS // 16):
            pltpu.sync_copy(zbuf, acc.at[pl.ds(sid * RPS + t * 16, 16)])
        pltpu.sync_copy(pidx_hbm.at[wid], pidx)
        plsc.subcore_barrier()

        def gath(j, b):
            _unpack_batch(pidx, j, sbuf[b], dbuf[b])
            pltpu.async_copy(x_hbm.at[sbuf[b]], rows[b], gsem[b])

        def scat(j, b):
            pltpu.make_async_copy(x_hbm.at[sbuf[b]], rows[b], gsem[b]).wait()
            pltpu.sync_copy(rows[b], acc.at[dbuf[b]], add=True)

        gath(0, 0)

        @pl.loop(0, KB // 2 - 1)
        def _(g):
            j0 = g * 2
            gath(j0 + 1, 1)
            scat(j0, 0)
            gath(j0 + 2, 0)
            scat(j0 + 1, 1)

        gath(KB - 1, 1)
        scat(KB - 2, 0)
        scat(KB - 1, 1)

        plsc.subcore_barrier()

        @pl.when(cid == 0)
        def _():
            pltpu.sync_copy(acc.at[pl.ds(sid * RPS, RPS)],
                            out0.at[pl.ds(sid * RPS, RPS)])

        @pl.when(cid == 1)
        def _():
            pltpu.sync_copy(acc.at[pl.ds(sid * RPS, RPS)],
                            out1.at[pl.ds(sid * RPS, RPS)])

    return agg


_agg_d = _make_agg_kernel(D)
_agg_c = _make_agg_kernel(CP)


_R = 512
_GRID = NPAD // _R


def _scale_in_body(x_ref, d0_ref, d1_ref, xt_ref, rdeg_ref):
    deg = jnp.maximum(d0_ref[...] + d1_ref[...], 1.0)
    rd = lax.rsqrt(deg)
    rdeg_ref[...] = rd
    xt_ref[...] = x_ref[...] * rd


def _scale_in(x_pad, deg0, deg1):
    return pl.pallas_call(
        _scale_in_body,
        grid=(_GRID,),
        in_specs=[
            pl.BlockSpec((_R, D), lambda i: (i, 0)),
            pl.BlockSpec((_R, 1), lambda i: (i, 0)),
            pl.BlockSpec((_R, 1), lambda i: (i, 0)),
        ],
        out_specs=[
            pl.BlockSpec((_R, D), lambda i: (i, 0)),
            pl.BlockSpec((_R, 1), lambda i: (i, 0)),
        ],
        out_shape=[
            jax.ShapeDtypeStruct((NPAD, D), jnp.float32),
            jax.ShapeDtypeStruct((NPAD, 1), jnp.float32),
        ],
    )(x_pad, deg0, deg1)


def _mid_body(a0_ref, a1_ref, rd_ref, w1_ref, b1_ref, w2_ref, yt_ref):
    rd = rd_ref[...]
    a = (a0_ref[...] + a1_ref[...]) * rd
    z = jnp.dot(a, w1_ref[...], preferred_element_type=jnp.float32)
    z = z + b1_ref[...]
    h = jnp.where(z > 0, z, jnp.exp(z) - 1.0)
    yt_ref[...] = jnp.dot(h * rd, w2_ref[...],
                          preferred_element_type=jnp.float32)


def _mid(a0, a1, rdeg, W1, b1, W2p):
    return pl.pallas_call(
        _mid_body,
        grid=(_GRID,),
        in_specs=[
            pl.BlockSpec((_R, D), lambda i: (i, 0)),
            pl.BlockSpec((_R, D), lambda i: (i, 0)),
            pl.BlockSpec((_R, 1), lambda i: (i, 0)),
            pl.BlockSpec((D, H), lambda i: (0, 0)),
            pl.BlockSpec((1, H), lambda i: (0, 0)),
            pl.BlockSpec((H, CP), lambda i: (0, 0)),
        ],
        out_specs=pl.BlockSpec((_R, CP), lambda i: (i, 0)),
        out_shape=jax.ShapeDtypeStruct((NPAD, CP), jnp.float32),
    )(a0, a1, rdeg, W1, b1, W2p)


def _scale_out_body(q0_ref, q1_ref, rd_ref, b2_ref, out_ref):
    out_ref[...] = (q0_ref[...] + q1_ref[...]) * rd_ref[...] + b2_ref[...]


def _scale_out(q0, q1, rdeg, b2p):
    return pl.pallas_call(
        _scale_out_body,
        grid=(_GRID,),
        in_specs=[
            pl.BlockSpec((_R, CP), lambda i: (i, 0)),
            pl.BlockSpec((_R, CP), lambda i: (i, 0)),
            pl.BlockSpec((_R, 1), lambda i: (i, 0)),
            pl.BlockSpec((1, CP), lambda i: (0, 0)),
        ],
        out_specs=pl.BlockSpec((_R, CP), lambda i: (i, 0)),
        out_shape=jax.ShapeDtypeStruct((NPAD, CP), jnp.float32),
    )(q0, q1, rdeg, b2p)


@jax.jit
def kernel(features, edge_index, W1, b1, W2, b2):
    epad = EPAD - E
    pad_nodes = N + jax.lax.rem(jnp.arange(epad, dtype=jnp.int32),
                                jnp.int32(NPAD - N))
    src_flat = jnp.concatenate([edge_index[0], pad_nodes])
    dst_flat = jnp.concatenate([edge_index[1], pad_nodes])
    packed = (src_flat | (dst_flat << 14)).reshape(NTILE, KB, BB)
    dst32 = dst_flat.reshape(NTILE, KB, BB)
    x_pad = jnp.pad(features, ((0, NPAD - N), (0, 0)))
    W2p = jnp.pad(W2, ((0, 0), (0, CP - C)))
    b1r = b1.reshape(1, H)
    b2p = jnp.pad(b2, (0, CP - C)).reshape(1, CP)

    deg0, deg1 = _deg_kernel(dst32)
    xt, rdeg = _scale_in(x_pad, deg0.reshape(NPAD, 1), deg1.reshape(NPAD, 1))
    a0, a1 = _agg_d(packed, xt)
    yt = _mid(a0, a1, rdeg, W1, b1r, W2p)
    q0, q1 = _agg_c(packed, yt)
    out = _scale_out(q0, q1, rdeg, b2p)
    return out[:N, :C]


# speedup vs baseline: 2.9629x; 1.0826x over previous
import functools

import jax
import jax.numpy as jnp
from jax import lax
from jax.experimental import pallas as pl
from jax.experimental.pallas import tpu as pltpu
from jax.experimental.pallas import tpu_sc as plsc

N = 10000
E = 320000
D = 128
H = 256
C = 47
CP = 48

NPAD = 10240
NTILE = 32
EPAD = 327680
EPT = EPAD // NTILE
BB = 128
KB = EPT // BB
RPS = NPAD // 16

_mesh = plsc.VectorSubcoreMesh(core_axis_name="c", subcore_axis_name="s")


def _zero_vmem_2d(zbuf, rows, cols):
    z16 = jnp.zeros((16,), jnp.float32)
    for r in range(rows):
        for c in range(cols // 16):
            zbuf[r, pl.ds(c * 16, 16)] = z16


def _unpack_batch(packed, j, sbuf, dbuf):
    mask = jnp.full((16,), 0x3FFF, jnp.int32)
    for k in range(BB // 16):
        v = packed[j, pl.ds(k * 16, 16)]
        sbuf[pl.ds(k * 16, 16)] = v & mask
        dbuf[pl.ds(k * 16, 16)] = lax.shift_right_logical(v, 14)


@functools.partial(
    pl.kernel,
    mesh=_mesh,
    out_type=(
        jax.ShapeDtypeStruct((NPAD,), jnp.float32),
        jax.ShapeDtypeStruct((NPAD,), jnp.float32),
    ),
    scratch_types=[
        pltpu.VMEM((KB, BB), jnp.int32),
        pltpu.VMEM((BB,), jnp.float32),
        pltpu.VMEM((RPS,), jnp.float32),
        pltpu.VMEM_SHARED((NPAD,), jnp.float32),
        pltpu.SemaphoreType.DMA,
    ],
    compiler_params=pltpu.CompilerParams(disable_bounds_checks=True),
)
def _deg_kernel(dst_hbm, out0, out1, didx, ones_v, zrow, acc, sem):
    cid = lax.axis_index("c")
    sid = lax.axis_index("s")
    wid = cid * 16 + sid
    for i in range(BB // 16):
        ones_v[pl.ds(i * 16, 16)] = jnp.ones((16,), jnp.float32)
    for i in range(RPS // 16):
        zrow[pl.ds(i * 16, 16)] = jnp.zeros((16,), jnp.float32)
    pltpu.sync_copy(dst_hbm.at[wid], didx)
    pltpu.sync_copy(zrow, acc.at[pl.ds(sid * RPS, RPS)])
    plsc.subcore_barrier()

    @pl.loop(0, KB)
    def _(j):
        pltpu.async_copy(ones_v, acc.at[didx.at[j]], sem, add=True)

    @pl.loop(0, KB)
    def _(j):
        pltpu.make_async_copy(ones_v, acc.at[didx.at[j]], sem).wait()

    plsc.subcore_barrier()

    @pl.when(cid == 0)
    def _():
        pltpu.sync_copy(acc.at[pl.ds(sid * RPS, RPS)],
                        out0.at[pl.ds(sid * RPS, RPS)])

    @pl.when(cid == 1)
    def _():
        pltpu.sync_copy(acc.at[pl.ds(sid * RPS, RPS)],
                        out1.at[pl.ds(sid * RPS, RPS)])


def _make_agg_kernel(width):
    @functools.partial(
        pl.kernel,
        mesh=_mesh,
        out_type=(
            jax.ShapeDtypeStruct((NPAD, width), jnp.float32),
            jax.ShapeDtypeStruct((NPAD, width), jnp.float32),
        ),
        scratch_types=[
            pltpu.VMEM((KB, BB), jnp.int32),
            [pltpu.VMEM((BB,), jnp.int32)] * 2,
            [pltpu.VMEM((BB,), jnp.int32)] * 2,
            [pltpu.VMEM((BB, width), jnp.float32)] * 2,
            pltpu.VMEM((16, width), jnp.float32),
            pltpu.VMEM_SHARED((NPAD, width), jnp.float32),
            [pltpu.SemaphoreType.DMA] * 2,
        ],
        compiler_params=pltpu.CompilerParams(
            use_tc_tiling_on_sc=(width % 128 == 0),
            disable_bounds_checks=True),
    )
    def agg(pidx_hbm, x_hbm, out0, out1, pidx, sbuf, dbuf, rows, zbuf,
            acc, gsem):
        cid = lax.axis_index("c")
        sid = lax.axis_index("s")
        wid = cid * 16 + sid
        _zero_vmem_2d(zbuf, 16, width)
        for t in range(RPS // 16):
            pltpu.sync_copy(zbuf, acc.at[pl.ds(sid * RPS + t * 16, 16)])
        pltpu.sync_copy(pidx_hbm.at[wid], pidx)
        plsc.subcore_barrier()

        def gath(j, b):
            _unpack_batch(pidx, j, sbuf[b], dbuf[b])
            pltpu.async_copy(x_hbm.at[sbuf[b]], rows[b], gsem[b])

        def scat(j, b):
            pltpu.make_async_copy(x_hbm.at[sbuf[b]], rows[b], gsem[b]).wait()
            pltpu.sync_copy(rows[b], acc.at[dbuf[b]], add=True)

        gath(0, 0)

        @pl.loop(0, KB // 2 - 1)
        def _(g):
            j0 = g * 2
            gath(j0 + 1, 1)
            scat(j0, 0)
            gath(j0 + 2, 0)
            scat(j0 + 1, 1)

        gath(KB - 1, 1)
        scat(KB - 2, 0)
        scat(KB - 1, 1)

        plsc.subcore_barrier()

        @pl.when(cid == 0)
        def _():
            pltpu.sync_copy(acc.at[pl.ds(sid * RPS, RPS)],
                            out0.at[pl.ds(sid * RPS, RPS)])

        @pl.when(cid == 1)
        def _():
            pltpu.sync_copy(acc.at[pl.ds(sid * RPS, RPS)],
                            out1.at[pl.ds(sid * RPS, RPS)])

    return agg


_agg_d = _make_agg_kernel(D)
_agg_c = _make_agg_kernel(CP)


_R = 2048
_GRID = NPAD // _R


def _scale_in_body(x_ref, d0_ref, d1_ref, xt_ref, rdeg_ref):
    deg = jnp.maximum(d0_ref[...] + d1_ref[...], 1.0)
    rd = lax.rsqrt(deg)
    rdeg_ref[...] = rd
    xt_ref[...] = x_ref[...] * rd


def _scale_in(x_pad, deg0, deg1):
    return pl.pallas_call(
        _scale_in_body,
        grid=(_GRID,),
        in_specs=[
            pl.BlockSpec((_R, D), lambda i: (i, 0)),
            pl.BlockSpec((_R, 1), lambda i: (i, 0)),
            pl.BlockSpec((_R, 1), lambda i: (i, 0)),
        ],
        out_specs=[
            pl.BlockSpec((_R, D), lambda i: (i, 0)),
            pl.BlockSpec((_R, 1), lambda i: (i, 0)),
        ],
        out_shape=[
            jax.ShapeDtypeStruct((NPAD, D), jnp.float32),
            jax.ShapeDtypeStruct((NPAD, 1), jnp.float32),
        ],
    )(x_pad, deg0, deg1)


def _mid_body(a0_ref, a1_ref, rd_ref, w1_ref, b1_ref, w2_ref, yt_ref):
    rd = rd_ref[...]
    a = (a0_ref[...] + a1_ref[...]) * rd
    z = jnp.dot(a, w1_ref[...], preferred_element_type=jnp.float32)
    z = z + b1_ref[...]
    h = jnp.where(z > 0, z, jnp.exp(z) - 1.0)
    yt_ref[...] = jnp.dot(h * rd, w2_ref[...],
                          preferred_element_type=jnp.float32)


def _mid(a0, a1, rdeg, W1, b1, W2p):
    return pl.pallas_call(
        _mid_body,
        grid=(_GRID,),
        in_specs=[
            pl.BlockSpec((_R, D), lambda i: (i, 0)),
            pl.BlockSpec((_R, D), lambda i: (i, 0)),
            pl.BlockSpec((_R, 1), lambda i: (i, 0)),
            pl.BlockSpec((D, H), lambda i: (0, 0)),
            pl.BlockSpec((1, H), lambda i: (0, 0)),
            pl.BlockSpec((H, CP), lambda i: (0, 0)),
        ],
        out_specs=pl.BlockSpec((_R, CP), lambda i: (i, 0)),
        out_shape=jax.ShapeDtypeStruct((NPAD, CP), jnp.float32),
    )(a0, a1, rdeg, W1, b1, W2p)


def _scale_out_body(q0_ref, q1_ref, rd_ref, b2_ref, out_ref):
    out_ref[...] = (q0_ref[...] + q1_ref[...]) * rd_ref[...] + b2_ref[...]


def _scale_out(q0, q1, rdeg, b2p):
    return pl.pallas_call(
        _scale_out_body,
        grid=(_GRID,),
        in_specs=[
            pl.BlockSpec((_R, CP), lambda i: (i, 0)),
            pl.BlockSpec((_R, CP), lambda i: (i, 0)),
            pl.BlockSpec((_R, 1), lambda i: (i, 0)),
            pl.BlockSpec((1, CP), lambda i: (0, 0)),
        ],
        out_specs=pl.BlockSpec((_R, CP), lambda i: (i, 0)),
        out_shape=jax.ShapeDtypeStruct((NPAD, CP), jnp.float32),
    )(q0, q1, rdeg, b2p)


@jax.jit
def kernel(features, edge_index, W1, b1, W2, b2):
    epad = EPAD - E
    pad_nodes = N + jax.lax.rem(jnp.arange(epad, dtype=jnp.int32),
                                jnp.int32(NPAD - N))
    src_flat = jnp.concatenate([edge_index[0], pad_nodes])
    dst_flat = jnp.concatenate([edge_index[1], pad_nodes])
    packed = (src_flat | (dst_flat << 14)).reshape(NTILE, KB, BB)
    dst32 = dst_flat.reshape(NTILE, KB, BB)
    x_pad = jnp.pad(features, ((0, NPAD - N), (0, 0)))
    W2p = jnp.pad(W2, ((0, 0), (0, CP - C)))
    b1r = b1.reshape(1, H)
    b2p = jnp.pad(b2, (0, CP - C)).reshape(1, CP)

    deg0, deg1 = _deg_kernel(dst32)
    xt, rdeg = _scale_in(x_pad, deg0.reshape(NPAD, 1), deg1.reshape(NPAD, 1))
    a0, a1 = _agg_d(packed, xt)
    yt = _mid(a0, a1, rdeg, W1, b1r, W2p)
    q0, q1 = _agg_c(packed, yt)
    out = _scale_out(q0, q1, rdeg, b2p)
    return out[:N, :C]
